# Initial kernel scaffold; baseline (speedup 1.0000x reference)
import jax
import jax.numpy as jnp
from jax.experimental import pallas as pl


def kernel(s_ctx, f_ctx, s_test):
    raise NotImplementedError("write your pallas kernel here")


# speedup vs baseline: 17.7924x; 17.7924x over previous
import functools

import jax
import jax.numpy as jnp
from jax.experimental import pallas as pl
from jax.experimental.pallas import tpu as pltpu

_K_NEIGHBORS = 10
_MASKED = 3.0e38


def _knn_body(q_ref, k_ref, f_ref, o_ref, *, softmax: bool):
    x = q_ref[0]
    y = k_ref[0]
    f = f_ref[0]
    R = x.shape[0]
    K = y.shape[0]

    x2 = jnp.sum(x * x, axis=-1, keepdims=True)
    y2 = jnp.sum(y * y, axis=-1)[None, :]
    xy = jax.lax.dot_general(
        x, y, (((1,), (1,)), ((), ())),
        precision=jax.lax.Precision.DEFAULT,
        preferred_element_type=jnp.float32,
    )
    dist = x2 + y2 - 2.0 * xy

    col = jax.lax.broadcasted_iota(jnp.int32, (R, K), 1)
    weights = jnp.zeros((R, K), jnp.float32)
    norm = jnp.zeros((R, 1), jnp.float32)
    m0 = None
    w = None
    for i in range(_K_NEIGHBORS):
        m = jnp.min(dist, axis=1, keepdims=True)
        if softmax:
            if i == 0:
                m0 = m
                w = jnp.ones((R, 1), jnp.float32)
            else:
                w = jnp.exp(m0 - m)
            norm = norm + w
        idx = jnp.where(dist == m, col, K)
        cmin = jnp.min(idx, axis=1, keepdims=True)
        sel = idx == cmin
        weights = jnp.where(sel, w if softmax else 1.0, weights)
        dist = jnp.where(sel, _MASKED, dist)

    out = jax.lax.dot_general(
        weights, f, (((1,), (0,)), ((), ())),
        precision=jax.lax.Precision.HIGHEST,
        preferred_element_type=jnp.float32,
    )
    if softmax:
        o_ref[0] = out / norm
    else:
        o_ref[0] = out * (1.0 / _K_NEIGHBORS)


def _knn_stage(q, k, f, softmax: bool):
    B, N, d = q.shape
    _, K, F = f.shape
    row_block = min(256, N)
    grid = (B, N // row_block)
    return pl.pallas_call(
        functools.partial(_knn_body, softmax=softmax),
        grid=grid,
        in_specs=[
            pl.BlockSpec((1, row_block, d), lambda b, t: (b, t, 0)),
            pl.BlockSpec((1, K, d), lambda b, t: (b, 0, 0)),
            pl.BlockSpec((1, K, F), lambda b, t: (b, 0, 0)),
        ],
        out_specs=pl.BlockSpec((1, row_block, F), lambda b, t: (b, t, 0)),
        out_shape=jax.ShapeDtypeStruct((B, N, F), jnp.float32),
        compiler_params=pltpu.CompilerParams(
            dimension_semantics=("arbitrary", "arbitrary"),
        ),
    )(q, k, f)


def kernel(s_ctx, f_ctx, s_test):
    f_ctx_s = _knn_stage(s_ctx, s_ctx, f_ctx, softmax=False)
    return _knn_stage(s_test, s_ctx, f_ctx_s, softmax=True)
